# baseline (speedup 1.0000x reference)
import jax
import jax.numpy as jnp
from jax import lax
from jax.experimental import pallas as pl
from jax.experimental.pallas import tpu as pltpu
from jax.experimental.pallas import tpu_sc as plsc

B = 16384
R_IN = 26
R_OUT = 11
D = 64
RUNS = ((0, 2, 0), (6, 1, 2), (12, 4, 3), (17, 1, 7), (20, 3, 8))

NC, NS = 2, 16
NW = NC * NS
BPW = B // NW

W_IN = R_IN * D
W_OUT = R_OUT * D


def _sc_body(in_hbm, out_hbm, sem):
    wid = lax.axis_index("s") * NC + lax.axis_index("c")
    b0 = wid * BPW
    copies = [
        pltpu.async_copy(
            in_hbm.at[pl.ds(b0, BPW), pl.ds(src * D, w * D)],
            out_hbm.at[pl.ds(b0, BPW), pl.ds(dst * D, w * D)],
            sem,
        )
        for (src, w, dst) in RUNS
    ]
    for h in copies:
        h.wait()


@jax.jit
def kernel(inputs):
    in2 = inputs.reshape(B, W_IN)
    mesh = plsc.VectorSubcoreMesh(core_axis_name="c", subcore_axis_name="s")
    out2 = pl.kernel(
        _sc_body,
        out_type=jax.ShapeDtypeStruct((B, W_OUT), jnp.float32),
        mesh=mesh,
        scratch_types=[pltpu.SemaphoreType.DMA],
        compiler_params=pltpu.CompilerParams(use_tc_tiling_on_sc=False),
    )(in2)
    return out2.reshape(B, R_OUT, D)


# speedup vs baseline: 5.2359x; 5.2359x over previous
import jax
import jax.numpy as jnp
from jax import lax
from jax.experimental import pallas as pl
from jax.experimental.pallas import tpu as pltpu
from jax.experimental.pallas import tpu_sc as plsc

B = 16384
R_IN = 26
R_OUT = 11
D = 64
RUNS = ((0, 2, 0), (6, 1, 2), (12, 4, 3), (17, 1, 7), (20, 3, 8))

NC, NS = 2, 16
NW = NC * NS
BPW = B // NW
NB = 64
NCHUNK = BPW // NB
NSLOT = 2
AHEAD = NSLOT - 1

W_IN = R_IN * D
W_OUT = R_OUT * D


def _sc_body(in_hbm, out_hbm, buf, *sems):
    rsems, wsems = sems[:NSLOT], sems[NSLOT:]
    wid = lax.axis_index("s") * NC + lax.axis_index("c")
    base = wid * BPW
    reads = [None] * NSLOT
    writes = [None] * NSLOT
    for c in range(NCHUNK + AHEAD):
        if c < NCHUNK:
            s = c % NSLOT
            if writes[s] is not None:
                writes[s].wait()
            b0 = base + c * NB
            reads[s] = [
                pltpu.async_copy(
                    in_hbm.at[pl.ds(b0, NB), pl.ds(src * D, w * D)],
                    buf.at[s, :, pl.ds(dst * D, w * D)],
                    rsems[s],
                )
                for (src, w, dst) in RUNS
            ]
        d = c - AHEAD
        if 0 <= d < NCHUNK:
            s = d % NSLOT
            for h in reads[s]:
                h.wait()
            writes[s] = pltpu.async_copy(
                buf.at[s], out_hbm.at[pl.ds(base + d * NB, NB)], wsems[s]
            )
    for h in writes:
        if h is not None:
            h.wait()


@jax.jit
def kernel(inputs):
    in2 = inputs.reshape(B, W_IN)
    mesh = plsc.VectorSubcoreMesh(core_axis_name="c", subcore_axis_name="s")
    out2 = pl.kernel(
        _sc_body,
        out_type=jax.ShapeDtypeStruct((B, W_OUT), jnp.float32),
        mesh=mesh,
        scratch_types=[pltpu.VMEM((NSLOT, NB, W_OUT), jnp.float32)]
        + [pltpu.SemaphoreType.DMA] * (2 * NSLOT),
        compiler_params=pltpu.CompilerParams(use_tc_tiling_on_sc=False),
    )(in2)
    return out2.reshape(B, R_OUT, D)
